# Initial kernel scaffold; baseline (speedup 1.0000x reference)
import jax
import jax.numpy as jnp
from jax.experimental import pallas as pl


def kernel(inputs, emb_highway, emb_length, emb_radian, emb_lon, emb_lat, emb_lanes, emb_c_centrality, emb_b_centrality, emb_h_centrality, emb_degree):
    raise NotImplementedError("write your pallas kernel here")


# speedup vs baseline: 4.2735x; 4.2735x over previous
import functools

import jax
import jax.numpy as jnp
from jax import lax
from jax.experimental import pallas as pl
from jax.experimental.pallas import tpu as pltpu
from jax.experimental.pallas import tpu_sc as plsc

N = 16384
OUT_D = 256
NUM_WORKERS = 32
ROWS_PER_W = N // NUM_WORKERS
CHUNK = 128
NCHUNK = ROWS_PER_W // CHUNK

_LOOKUPS = (
    (0, 0, 0, 16),
    (1, 1, 16, 16),
    (2, 2, 32, 16),
    (3, 3, 48, 32),
    (4, 4, 80, 32),
    (3, 5, 112, 32),
    (4, 6, 144, 32),
    (5, 7, 176, 16),
    (6, 8, 192, 16),
    (7, 9, 208, 16),
    (8, 10, 224, 16),
    (9, 11, 240, 16),
)

_mesh = plsc.VectorSubcoreMesh(core_axis_name="c", subcore_axis_name="s")


@functools.partial(
    pl.kernel,
    mesh=_mesh,
    compiler_params=pltpu.CompilerParams(use_tc_tiling_on_sc=False),
    out_type=jax.ShapeDtypeStruct((N, OUT_D), jnp.float32),
    scratch_types=(
        [pltpu.VMEM((12, CHUNK), jnp.int32)]
        + [pltpu.VMEM((CHUNK, d), jnp.float32) for (_, _, _, d) in _LOOKUPS]
        + [pltpu.SemaphoreType.DMA, pltpu.SemaphoreType.DMA]
    ),
)
def _emb_kernel(idx_hbm, t0, t1, t2, t3, t4, t5, t6, t7, t8, t9, out_hbm,
                idx_v, b0, b1, b2, b3, b4, b5, b6, b7, b8, b9, b10, b11,
                gsem, wsem):
    tables = (t0, t1, t2, t3, t4, t5, t6, t7, t8, t9)
    bufs = (b0, b1, b2, b3, b4, b5, b6, b7, b8, b9, b10, b11)
    wid = lax.axis_index("s") * 2 + lax.axis_index("c")
    base = wid * ROWS_PER_W
    for c in range(NCHUNK):
        cb = base + c * CHUNK
        pltpu.sync_copy(idx_hbm.at[:, pl.ds(cb, CHUNK)], idx_v)
        gh = []
        for j, (t, col, _, _) in enumerate(_LOOKUPS):
            gh.append(pltpu.async_copy(tables[t].at[idx_v.at[col]], bufs[j], gsem))
        wh = []
        for j, (_, _, off, d) in enumerate(_LOOKUPS):
            gh[j].wait()
            wh.append(pltpu.async_copy(
                bufs[j], out_hbm.at[pl.ds(cb, CHUNK), pl.ds(off, d)], wsem))
        for h in wh:
            h.wait()


def kernel(inputs, emb_highway, emb_length, emb_radian, emb_lon, emb_lat,
           emb_lanes, emb_c_centrality, emb_b_centrality, emb_h_centrality,
           emb_degree):
    idx_t = inputs[:, 2:14].T
    return _emb_kernel(idx_t, emb_highway, emb_length, emb_radian, emb_lon,
                       emb_lat, emb_lanes, emb_c_centrality, emb_b_centrality,
                       emb_h_centrality, emb_degree)


# speedup vs baseline: 4.3680x; 1.0221x over previous
import functools

import jax
import jax.numpy as jnp
from jax import lax
from jax.experimental import pallas as pl
from jax.experimental.pallas import tpu as pltpu
from jax.experimental.pallas import tpu_sc as plsc

N = 16384
OUT_D = 256
NUM_WORKERS = 32
ROWS_PER_W = N // NUM_WORKERS
CHUNK = 128
NCHUNK = ROWS_PER_W // CHUNK

_LOOKUPS = (
    (0, 0, 0, 16),
    (1, 1, 16, 16),
    (2, 2, 32, 16),
    (3, 3, 48, 32),
    (4, 4, 80, 32),
    (3, 5, 112, 32),
    (4, 6, 144, 32),
    (5, 7, 176, 16),
    (6, 8, 192, 16),
    (7, 9, 208, 16),
    (8, 10, 224, 16),
    (9, 11, 240, 16),
)

_mesh = plsc.VectorSubcoreMesh(core_axis_name="c", subcore_axis_name="s")


@functools.partial(
    pl.kernel,
    mesh=_mesh,
    compiler_params=pltpu.CompilerParams(use_tc_tiling_on_sc=False),
    out_type=jax.ShapeDtypeStruct((N, OUT_D), jnp.float32),
    scratch_types=(
        [pltpu.VMEM((12, ROWS_PER_W), jnp.int32)]
        + [pltpu.VMEM((CHUNK, d), jnp.float32)
           for _ in range(2) for (_, _, _, d) in _LOOKUPS]
        + [pltpu.SemaphoreType.DMA for _ in range(4)]
    ),
)
def _emb_kernel(idx_hbm, t0, t1, t2, t3, t4, t5, t6, t7, t8, t9, out_hbm,
                idx_v, *rest):
    tables = (t0, t1, t2, t3, t4, t5, t6, t7, t8, t9)
    bufs = (rest[0:12], rest[12:24])
    gsems = (rest[24], rest[25])
    wsems = (rest[26], rest[27])
    wid = lax.axis_index("s") * 2 + lax.axis_index("c")
    base = wid * ROWS_PER_W
    pltpu.sync_copy(idx_hbm.at[:, pl.ds(base, ROWS_PER_W)], idx_v)

    def fire_gathers(c):
        hs = []
        for j, (t, col, _, _) in enumerate(_LOOKUPS):
            hs.append(pltpu.async_copy(
                tables[t].at[idx_v.at[col, pl.ds(c * CHUNK, CHUNK)]],
                bufs[c % 2][j],
                gsems[c % 2]))
        return hs

    def fire_writes(c):
        hs = []
        for j, (_, _, off, d) in enumerate(_LOOKUPS):
            hs.append(pltpu.async_copy(
                bufs[c % 2][j],
                out_hbm.at[pl.ds(base + c * CHUNK, CHUNK), pl.ds(off, d)],
                wsems[c % 2]))
        return hs

    ghs = [None, None]
    whs = [None, None]
    ghs[0] = fire_gathers(0)
    for c in range(NCHUNK):
        if c + 1 < NCHUNK:
            if whs[(c + 1) % 2] is not None:
                for h in whs[(c + 1) % 2]:
                    h.wait()
            ghs[(c + 1) % 2] = fire_gathers(c + 1)
        for h in ghs[c % 2]:
            h.wait()
        whs[c % 2] = fire_writes(c)
    for p in (0, 1):
        if whs[p] is not None:
            for h in whs[p]:
                h.wait()


def kernel(inputs, emb_highway, emb_length, emb_radian, emb_lon, emb_lat,
           emb_lanes, emb_c_centrality, emb_b_centrality, emb_h_centrality,
           emb_degree):
    idx_t = inputs[:, 2:14].T
    return _emb_kernel(idx_t, emb_highway, emb_length, emb_radian, emb_lon,
                       emb_lat, emb_lanes, emb_c_centrality, emb_b_centrality,
                       emb_h_centrality, emb_degree)
